# baseline (speedup 1.0000x reference)
import jax
import jax.numpy as jnp
from jax import lax
from jax.experimental import pallas as pl
from jax.experimental.pallas import tpu as pltpu
from jax.experimental.pallas import tpu_sc as plsc

VOCAB = 8192
D = 8192
B = 8192
NW = 32
B_PER_W = B // NW
CH = 4
NCHUNK = B_PER_W // CH
NBUF = 3
NTRIPLE = (NCHUNK - 1) // NBUF


def _gather_body(idx_hbm, table_hbm, out_hbm, idx_v, rows_v,
                 g0, g1, g2, s0, s1, s2):
    cid = lax.axis_index("c")
    sid = lax.axis_index("s")
    wid = sid * 2 + cid
    base = wid * B_PER_W
    gsem = (g0, g1, g2)
    ssem = (s0, s1, s2)

    pltpu.sync_copy(idx_hbm.at[wid], idx_v)

    def gather(c, buf, sem):
        return pltpu.make_async_copy(
            table_hbm.at[idx_v.at[c]], rows_v.at[buf], sem)

    def scatter(c, buf, sem):
        return pltpu.make_async_copy(
            rows_v.at[buf], out_hbm.at[pl.ds(base + c * CH, CH)], sem)

    for b in range(NBUF):
        gather(b, b, gsem[b]).start()

    def triple(i, carry):
        for b in range(NBUF):
            c = NBUF * i + b
            gather(c, b, gsem[b]).wait()
            scatter(c, b, ssem[b]).start()
            nb = (b + 2) % NBUF

            @pl.when((c >= 1) & (c <= NCHUNK - 3))
            def _():
                scatter(c - 1, nb, ssem[nb]).wait()
                gather(c + 2, nb, gsem[nb]).start()
        return carry

    lax.fori_loop(0, NTRIPLE, triple, 0)

    c_last = NCHUNK - 1
    gather(c_last, 0, gsem[0]).wait()
    scatter(c_last, 0, ssem[0]).start()

    scatter(NCHUNK - 3, 1, ssem[1]).wait()
    scatter(NCHUNK - 2, 2, ssem[2]).wait()
    scatter(c_last, 0, ssem[0]).wait()


@jax.jit
def kernel(x, table):
    idx = x.reshape(NW, NCHUNK, CH).astype(jnp.int32)
    mesh = plsc.VectorSubcoreMesh(core_axis_name="c", subcore_axis_name="s")
    out = pl.kernel(
        _gather_body,
        mesh=mesh,
        out_type=jax.ShapeDtypeStruct((B, D), jnp.float32),
        scratch_types=[
            pltpu.VMEM((NCHUNK, CH), jnp.int32),
            pltpu.VMEM((NBUF, CH, D), jnp.float32),
            pltpu.SemaphoreType.DMA,
            pltpu.SemaphoreType.DMA,
            pltpu.SemaphoreType.DMA,
            pltpu.SemaphoreType.DMA,
            pltpu.SemaphoreType.DMA,
            pltpu.SemaphoreType.DMA,
        ],
    )(idx, table)
    return out.reshape(x.shape[0], x.shape[1], D)
